# baseline (speedup 1.0000x reference)
import functools

import jax
import jax.numpy as jnp
from jax import lax
from jax.experimental import pallas as pl
from jax.experimental.pallas import tpu as pltpu
from jax.experimental.pallas import tpu_sc as plsc

D = 768
H = 12
DH = 64
T = 8192
L = 2 * T
NCH = L // 64
NBKT = T // 64
NBIN = 2 * NBKT
SEG = L // 16
BT = 512

_SC_PARAMS = pltpu.CompilerParams(needs_layout_passes=False)
_sc_cache = {}


def _proj_body(x_ref, wqk_ref, wv_ref, rotc_ref, qkv_ref, bkt_ref):
    xb = x_ref[...]
    qk = jnp.dot(xb, wqk_ref[...], preferred_element_type=jnp.float32)
    v = jnp.dot(xb, wv_ref[...], preferred_element_type=jnp.float32)
    i = pl.program_id(0)
    tcol = (i * BT
            + lax.broadcasted_iota(jnp.int32, (BT, DH), 0)).astype(jnp.float32)
    zcol = jnp.zeros((BT, DH), jnp.float32)
    for h in range(H):
        qkh = qk[:, h * DH:(h + 1) * DH]
        qkv_ref[h, :, :] = jnp.concatenate(
            [qkh, v[:, h * DH:(h + 1) * DH], tcol, zcol], axis=1)
        for r in range(2):
            rot = jnp.dot(qkh, rotc_ref[r], preferred_element_type=jnp.float32)
            mx = jnp.max(rot, axis=1, keepdims=True)
            iota = lax.broadcasted_iota(jnp.int32, rot.shape, 1)
            cand = jnp.where(rot >= mx, iota, NBKT)
            bidx = jnp.min(cand, axis=1).astype(jnp.int32)
            bkt_ref[h * 2 + r, :] = bidx + r * NBKT


def _proj(x, wqk, wv, rotc):
    return pl.pallas_call(
        _proj_body,
        grid=(T // BT,),
        in_specs=[
            pl.BlockSpec((BT, D), lambda i: (i, 0)),
            pl.BlockSpec((D, D), lambda i: (0, 0)),
            pl.BlockSpec((D, D), lambda i: (0, 0)),
            pl.BlockSpec((2, DH, NBKT), lambda i: (0, 0, 0)),
        ],
        out_specs=[
            pl.BlockSpec((H, BT, 256), lambda i: (0, i, 0)),
            pl.BlockSpec((2 * H, BT), lambda i: (0, i)),
        ],
        out_shape=[
            jax.ShapeDtypeStruct((H, T, 256), jnp.float32),
            jax.ShapeDtypeStruct((2 * H, T), jnp.int32),
        ],
    )(x, wqk, wv, rotc)


def _sort_body(bkt_hbm, stick_hbm, bv, sv, cv, t16):
    wid = lax.axis_index("s") * 2 + lax.axis_index("c")
    lane = lax.iota(jnp.int32, 16)

    @pl.when(wid < H)
    def _():
        pltpu.sync_copy(bkt_hbm.at[wid], bv)

        def zero(i, _):
            cv[pl.ds(i * 16, 16)] = jnp.zeros((16,), jnp.int32)
            return 0
        lax.fori_loop(0, NBIN, zero, 0)

        def hist(i, _):
            b = plsc.load_gather(bv, [lane * SEG + i])
            plsc.addupdate_scatter(cv, [lane * NBIN + b],
                                   jnp.ones((16,), jnp.int32))
            return 0
        lax.fori_loop(0, SEG, hist, 0)

        def prefix(vb, carry):
            c = plsc.load_gather(cv, [lane * NBIN + vb])
            s = c
            for sh in (1, 2, 4, 8):
                t16[...] = s
                g = plsc.load_gather(t16, [jnp.maximum(lane - sh, 0)])
                s = s + jnp.where(lane >= sh, g, jnp.zeros((16,), jnp.int32))
            t16[...] = s
            tot = plsc.load_gather(t16, [jnp.full((16,), 15, jnp.int32)])
            plsc.store_scatter(cv, [lane * NBIN + vb], s - c + carry)
            return carry + tot
        lax.fori_loop(0, NBIN, prefix, jnp.zeros((16,), jnp.int32))

        def scat(i, _):
            idx_el = lane * SEG + i
            b = plsc.load_gather(bv, [idx_el])
            cur = plsc.load_gather(cv, [lane * NBIN + b])
            plsc.store_scatter(sv, [cur], idx_el)
            plsc.store_scatter(cv, [lane * NBIN + b], cur + 1)
            return 0
        lax.fori_loop(0, SEG, scat, 0)

        pltpu.sync_copy(sv, stick_hbm.at[wid])


def _sort_sc(bkt):
    if "sort" not in _sc_cache:
        mesh = plsc.VectorSubcoreMesh(core_axis_name="c", subcore_axis_name="s")
        _sc_cache["sort"] = pl.kernel(
            _sort_body,
            out_type=[jax.ShapeDtypeStruct((H, L), jnp.int32)],
            mesh=mesh,
            compiler_params=_SC_PARAMS,
            scratch_types=[
                pltpu.VMEM((L,), jnp.int32),
                pltpu.VMEM((L,), jnp.int32),
                pltpu.VMEM((16 * NBIN,), jnp.int32),
                pltpu.VMEM((16,), jnp.int32),
            ],
        )
    return _sc_cache["sort"](bkt)


_GC = 256
_NJOB = 2 * H


def _gather_body(qkv_hbm, stick_hbm, sorted_hbm, sk, iv, rows, sem):
    wid = lax.axis_index("s") * 2 + lax.axis_index("c")
    _JL = L // 2

    @pl.when(wid < _NJOB)
    def _():
        h = wid // 2
        base = h * L + (wid % 2) * _JL

        def chunk(cix, _):
            off = base + cix * _GC
            pltpu.sync_copy(stick_hbm.at[pl.ds(off, _GC)], sk)

            def body16(i, _):
                s = sk[pl.ds(i * 16, 16)]
                iv[pl.ds(i * 16, 16)] = (s & (T - 1)) + h * T
                return 0
            lax.fori_loop(0, _GC // 16, body16, 0)
            pltpu.async_copy(qkv_hbm.at[iv], rows, sem).wait()
            pltpu.sync_copy(rows, sorted_hbm.at[pl.ds(off, _GC)])
            return 0
        lax.fori_loop(0, _JL // _GC, chunk, 0)


def _gather_sc(qkv_flat, stick_flat):
    if "gather" not in _sc_cache:
        mesh = plsc.VectorSubcoreMesh(core_axis_name="c", subcore_axis_name="s")
        _sc_cache["gather"] = pl.kernel(
            _gather_body,
            out_type=[
                jax.ShapeDtypeStruct((H * L, 256), jnp.float32),
            ],
            mesh=mesh,
            compiler_params=_SC_PARAMS,
            scratch_types=[
                pltpu.VMEM((_GC,), jnp.int32),
                pltpu.VMEM((_GC,), jnp.int32),
                pltpu.VMEM((_GC, 256), jnp.float32),
                pltpu.SemaphoreType.DMA,
            ],
        )
    return _sc_cache["gather"](qkv_flat, stick_flat)


_CG = 4
_QW = _CG * 64
_KW = _QW + 64


def _attn_body(sorted_ref, att_ref):
    def body(g, _):
        b0 = g * _QW
        p0 = (b0 + L - 64) % L
        cur = sorted_ref[0, pl.ds(b0, _QW), :]
        prv = sorted_ref[0, pl.ds(p0, 64), :]
        win = jnp.concatenate([prv, cur], axis=0)
        q = cur[:, 0:DH]
        kk = win[:, 0:DH]
        vv = win[:, DH:128]
        nk = kk / (jnp.sqrt(jnp.sum(kk * kk, axis=1, keepdims=True)) + 1e-9)
        dots = lax.dot_general(q, nk, (((1,), (1,)), ((), ())),
                               preferred_element_type=jnp.float32)
        dots = dots * (float(DH) ** -0.5)
        tq64 = cur[:, 128:192]
        tq = jnp.concatenate([tq64] * (_KW // DH), axis=1)
        tk = jnp.concatenate(
            [jnp.transpose(win[:, 128:192])] * (_QW // DH), axis=0)
        qc = lax.broadcasted_iota(jnp.int32, (_QW, _KW), 0) // 64
        kc = lax.broadcasted_iota(jnp.int32, (_QW, _KW), 1) // 64 - 1
        valid = (kc == qc) | (kc == qc - 1)
        dots = jnp.where(tq < tk, -1e9, dots)
        dots = jnp.where(tq == tk, -1e5, dots)
        dots = jnp.where(valid, dots, -1e30)
        m = jnp.max(dots, axis=1, keepdims=True)
        ex = jnp.exp(dots - m)
        se = jnp.sum(ex, axis=1, keepdims=True)
        lse = m + jnp.log(se)
        o = lax.dot_general(ex / se, vv, (((1,), (0,)), ((), ())),
                            preferred_element_type=jnp.float32)
        att_ref[0, pl.ds(b0, _QW), :] = jnp.concatenate(
            [o, jnp.broadcast_to(lse, (_QW, DH))], axis=1)
        return 0
    lax.fori_loop(0, NCH // _CG, body, 0)


def _attn(sorted_rows):
    return pl.pallas_call(
        _attn_body,
        grid=(H,),
        in_specs=[
            pl.BlockSpec((1, L, 256), lambda i: (i, 0, 0)),
        ],
        out_specs=pl.BlockSpec((1, L, 128), lambda i: (i, 0, 0)),
        out_shape=jax.ShapeDtypeStruct((H, L, 128), jnp.float32),
    )(sorted_rows)


def _scatter_body(att_hbm, stick_hbm, und_hbm, sk, iv, rows, sem):
    wid = lax.axis_index("s") * 2 + lax.axis_index("c")
    _JL = L // 2

    @pl.when(wid < _NJOB)
    def _():
        h = wid // 2
        base = h * L + (wid % 2) * _JL

        def chunk(cix, _):
            off = base + cix * _GC
            pltpu.sync_copy(stick_hbm.at[pl.ds(off, _GC)], sk)

            def body16(i, _):
                s = sk[pl.ds(i * 16, 16)]
                iv[pl.ds(i * 16, 16)] = s + h * L
                return 0
            lax.fori_loop(0, _GC // 16, body16, 0)
            pltpu.sync_copy(att_hbm.at[pl.ds(off, _GC)], rows)
            pltpu.async_copy(rows, und_hbm.at[iv], sem).wait()
            return 0
        lax.fori_loop(0, _JL // _GC, chunk, 0)


def _scatter_sc(att_flat, stick_flat):
    if "scatter" not in _sc_cache:
        mesh = plsc.VectorSubcoreMesh(core_axis_name="c", subcore_axis_name="s")
        _sc_cache["scatter"] = pl.kernel(
            _scatter_body,
            out_type=[jax.ShapeDtypeStruct((H * L, 128), jnp.float32)],
            mesh=mesh,
            compiler_params=_SC_PARAMS,
            scratch_types=[
                pltpu.VMEM((_GC,), jnp.int32),
                pltpu.VMEM((_GC,), jnp.int32),
                pltpu.VMEM((_GC, 128), jnp.float32),
                pltpu.SemaphoreType.DMA,
            ],
        )
    return _sc_cache["scatter"](att_flat, stick_flat)


def _layer_norm_rows(x, g, b):
    mu = jnp.mean(x, axis=-1, keepdims=True)
    xc = x - mu
    var = jnp.mean(xc * xc, axis=-1, keepdims=True)
    return xc / jnp.sqrt(var + 1e-6) * g + b


def _ffn_body(und_ref, x_ref, wo_ref, w1_ref, b1_ref, w2_ref, b2_ref,
              g1_ref, be1_ref, g2_ref, be2_ref, out_ref):
    pieces = []
    for h in range(H):
        u0 = und_ref[h, 0, :, :]
        u1 = und_ref[h, 1, :, :]
        o0 = u0[:, 0:DH]
        o1 = u1[:, 0:DH]
        l0 = u0[:, DH:128]
        l1 = u1[:, DH:128]
        m = jnp.maximum(l0, l1)
        e0 = jnp.exp(l0 - m)
        e1 = jnp.exp(l1 - m)
        pieces.append((e0 * o0 + e1 * o1) / (e0 + e1))
    attn = jnp.concatenate(pieces, axis=1)
    xb = x_ref[...]
    ao = jnp.dot(attn.astype(jnp.bfloat16), wo_ref[...],
                 preferred_element_type=jnp.float32) + xb
    h1 = _layer_norm_rows(ao, g1_ref[...], be1_ref[...])
    f = jnp.maximum(
        jnp.dot(h1.astype(jnp.bfloat16), w1_ref[...],
                preferred_element_type=jnp.float32)
        + b1_ref[...], 0.0)
    f2 = jnp.dot(f.astype(jnp.bfloat16), w2_ref[...],
                 preferred_element_type=jnp.float32) + b2_ref[...]
    out_ref[...] = _layer_norm_rows(f2 + h1, g2_ref[...], be2_ref[...])


def _ffn(und, x, wo, w1, b1, w2, b2, g1, be1, g2, be2):
    return pl.pallas_call(
        _ffn_body,
        grid=(T // BT,),
        in_specs=[
            pl.BlockSpec((H, 2, BT, 128), lambda i: (0, 0, i, 0)),
            pl.BlockSpec((BT, D), lambda i: (i, 0)),
            pl.BlockSpec((D, D), lambda i: (0, 0)),
            pl.BlockSpec((D, 3072), lambda i: (0, 0)),
            pl.BlockSpec((1, 3072), lambda i: (0, 0)),
            pl.BlockSpec((3072, D), lambda i: (0, 0)),
            pl.BlockSpec((1, D), lambda i: (0, 0)),
            pl.BlockSpec((1, D), lambda i: (0, 0)),
            pl.BlockSpec((1, D), lambda i: (0, 0)),
            pl.BlockSpec((1, D), lambda i: (0, 0)),
            pl.BlockSpec((1, D), lambda i: (0, 0)),
        ],
        out_specs=pl.BlockSpec((BT, D), lambda i: (i, 0)),
        out_shape=jax.ShapeDtypeStruct((T, D), jnp.float32),
    )(und, x, wo, w1, b1, w2, b2, g1, be1, g2, be2)


def kernel(enc_input, Wqk, Wv, Wo, rotations, ln1_g, ln1_b, W1, b1, W2, b2,
           ln2_g, ln2_b):
    x = enc_input.reshape(T, D)
    rotc = jnp.concatenate(
        [rotations.transpose(1, 0, 2), -rotations.transpose(1, 0, 2)],
        axis=-1)

    qkv, bkt = _proj(x, Wqk, Wv, rotc)
    (stick,) = _sort_sc(bkt.reshape(H, L))
    qkv_flat = qkv.reshape(H * T, 256)
    stick_flat = stick.reshape(H * L)
    (sorted_flat,) = _gather_sc(qkv_flat, stick_flat)
    att = _attn(sorted_flat.reshape(H, L, 256))
    (und_flat,) = _scatter_sc(att.reshape(H * L, 128), stick_flat)
    und = und_flat.reshape(H, 2, T, 128)
    out = _ffn(und, x, Wo.astype(jnp.bfloat16), W1.astype(jnp.bfloat16),
               b1.reshape(1, 3072), W2.astype(jnp.bfloat16), b2.reshape(1, D),
               ln1_g.reshape(1, D), ln1_b.reshape(1, D),
               ln2_g.reshape(1, D), ln2_b.reshape(1, D))
    return out.reshape(1, T, D)


# speedup vs baseline: 1.0104x; 1.0104x over previous
import functools

import jax
import jax.numpy as jnp
from jax import lax
from jax.experimental import pallas as pl
from jax.experimental.pallas import tpu as pltpu
from jax.experimental.pallas import tpu_sc as plsc

D = 768
H = 12
DH = 64
T = 8192
L = 2 * T
NCH = L // 64
NBKT = T // 64
NBIN = 2 * NBKT
SEG = L // 16
BT = 512

_SC_PARAMS = pltpu.CompilerParams(needs_layout_passes=False)
_sc_cache = {}


def _proj_body(x_ref, wqk_ref, wv_ref, rotc_ref, qkv_ref, bkt_ref):
    xb = x_ref[...]
    qk = jnp.dot(xb, wqk_ref[...], preferred_element_type=jnp.float32)
    v = jnp.dot(xb, wv_ref[...], preferred_element_type=jnp.float32)
    i = pl.program_id(0)
    tcol = (i * BT
            + lax.broadcasted_iota(jnp.int32, (BT, DH), 0)).astype(jnp.float32)
    zcol = jnp.zeros((BT, DH), jnp.float32)
    for h in range(H):
        qkh = qk[:, h * DH:(h + 1) * DH]
        qkv_ref[h, :, :] = jnp.concatenate(
            [qkh, v[:, h * DH:(h + 1) * DH], tcol, zcol], axis=1)
        for r in range(2):
            rot = jnp.dot(qkh, rotc_ref[r], preferred_element_type=jnp.float32)
            mx = jnp.max(rot, axis=1, keepdims=True)
            iota = lax.broadcasted_iota(jnp.int32, rot.shape, 1)
            cand = jnp.where(rot >= mx, iota, NBKT)
            bidx = jnp.min(cand, axis=1).astype(jnp.int32)
            bkt_ref[h * 2 + r, :] = bidx + r * NBKT


def _proj(x, wqk, wv, rotc):
    return pl.pallas_call(
        _proj_body,
        grid=(T // BT,),
        in_specs=[
            pl.BlockSpec((BT, D), lambda i: (i, 0)),
            pl.BlockSpec((D, D), lambda i: (0, 0)),
            pl.BlockSpec((D, D), lambda i: (0, 0)),
            pl.BlockSpec((2, DH, NBKT), lambda i: (0, 0, 0)),
        ],
        out_specs=[
            pl.BlockSpec((H, BT, 256), lambda i: (0, i, 0)),
            pl.BlockSpec((2 * H, BT), lambda i: (0, i)),
        ],
        out_shape=[
            jax.ShapeDtypeStruct((H, T, 256), jnp.float32),
            jax.ShapeDtypeStruct((2 * H, T), jnp.int32),
        ],
    )(x, wqk, wv, rotc)


def _sort_body(bkt_hbm, stick_hbm, bv, sv, cv, t16):
    wid = lax.axis_index("s") * 2 + lax.axis_index("c")
    lane = lax.iota(jnp.int32, 16)

    @pl.when(wid < H)
    def _():
        pltpu.sync_copy(bkt_hbm.at[wid], bv)

        def zero(i, _):
            cv[pl.ds(i * 16, 16)] = jnp.zeros((16,), jnp.int32)
            return 0
        lax.fori_loop(0, NBIN, zero, 0)

        def hist(i, _):
            b = plsc.load_gather(bv, [lane * SEG + i])
            plsc.addupdate_scatter(cv, [lane * NBIN + b],
                                   jnp.ones((16,), jnp.int32))
            return 0
        lax.fori_loop(0, SEG, hist, 0)

        def prefix(vb, carry):
            c = plsc.load_gather(cv, [lane * NBIN + vb])
            s = c
            for sh in (1, 2, 4, 8):
                t16[...] = s
                g = plsc.load_gather(t16, [jnp.maximum(lane - sh, 0)])
                s = s + jnp.where(lane >= sh, g, jnp.zeros((16,), jnp.int32))
            t16[...] = s
            tot = plsc.load_gather(t16, [jnp.full((16,), 15, jnp.int32)])
            plsc.store_scatter(cv, [lane * NBIN + vb], s - c + carry)
            return carry + tot
        lax.fori_loop(0, NBIN, prefix, jnp.zeros((16,), jnp.int32))

        def scat(i, _):
            idx_el = lane * SEG + i
            b = plsc.load_gather(bv, [idx_el])
            cur = plsc.load_gather(cv, [lane * NBIN + b])
            plsc.store_scatter(sv, [cur], idx_el)
            plsc.store_scatter(cv, [lane * NBIN + b], cur + 1)
            return 0
        lax.fori_loop(0, SEG, scat, 0)

        pltpu.sync_copy(sv, stick_hbm.at[wid])


def _sort_sc(bkt):
    if "sort" not in _sc_cache:
        mesh = plsc.VectorSubcoreMesh(core_axis_name="c", subcore_axis_name="s")
        _sc_cache["sort"] = pl.kernel(
            _sort_body,
            out_type=[jax.ShapeDtypeStruct((H, L), jnp.int32)],
            mesh=mesh,
            compiler_params=_SC_PARAMS,
            scratch_types=[
                pltpu.VMEM((L,), jnp.int32),
                pltpu.VMEM((L,), jnp.int32),
                pltpu.VMEM((16 * NBIN,), jnp.int32),
                pltpu.VMEM((16,), jnp.int32),
            ],
        )
    return _sc_cache["sort"](bkt)


_GC = 256
_NJOB = 2 * H


def _gather_body(qkv_hbm, stick_hbm, sorted_hbm, sk, iv, rows, sem):
    wid = lax.axis_index("s") * 2 + lax.axis_index("c")
    _JL = L // 2

    @pl.when(wid < _NJOB)
    def _():
        h = wid // 2
        base = h * L + (wid % 2) * _JL

        def chunk(cix, _):
            off = base + cix * _GC
            pltpu.sync_copy(stick_hbm.at[pl.ds(off, _GC)], sk)

            def body16(i, _):
                s = sk[pl.ds(i * 16, 16)]
                iv[pl.ds(i * 16, 16)] = (s & (T - 1)) + h * T
                return 0
            lax.fori_loop(0, _GC // 16, body16, 0)
            pltpu.async_copy(qkv_hbm.at[iv], rows, sem).wait()
            pltpu.sync_copy(rows, sorted_hbm.at[pl.ds(off, _GC)])
            return 0
        lax.fori_loop(0, _JL // _GC, chunk, 0)


def _gather_sc(qkv_flat, stick_flat):
    if "gather" not in _sc_cache:
        mesh = plsc.VectorSubcoreMesh(core_axis_name="c", subcore_axis_name="s")
        _sc_cache["gather"] = pl.kernel(
            _gather_body,
            out_type=[
                jax.ShapeDtypeStruct((H * L, 256), jnp.float32),
            ],
            mesh=mesh,
            compiler_params=_SC_PARAMS,
            scratch_types=[
                pltpu.VMEM((_GC,), jnp.int32),
                pltpu.VMEM((_GC,), jnp.int32),
                pltpu.VMEM((_GC, 256), jnp.float32),
                pltpu.SemaphoreType.DMA,
            ],
        )
    return _sc_cache["gather"](qkv_flat, stick_flat)


_CG = 4
_QW = _CG * 64
_KW = _QW + 64


def _attn_body(sorted_ref, att_ref):
    def body(g, _):
        b0 = g * _QW
        p0 = (b0 + L - 64) % L
        cur = sorted_ref[0, pl.ds(b0, _QW), :]
        prv = sorted_ref[0, pl.ds(p0, 64), :]
        win = jnp.concatenate([prv, cur], axis=0)
        q = cur[:, 0:DH]
        kk = win[:, 0:DH]
        vv = win[:, DH:128]
        nk = kk / (jnp.sqrt(jnp.sum(kk * kk, axis=1, keepdims=True)) + 1e-9)
        dots = lax.dot_general(q, nk, (((1,), (1,)), ((), ())),
                               preferred_element_type=jnp.float32)
        dots = dots * (float(DH) ** -0.5)
        tq64 = cur[:, 128:192]
        tq = jnp.concatenate([tq64] * (_KW // DH), axis=1)
        tk = jnp.concatenate(
            [jnp.transpose(win[:, 128:192])] * (_QW // DH), axis=0)
        qc = lax.broadcasted_iota(jnp.int32, (_QW, _KW), 0) // 64
        kc = lax.broadcasted_iota(jnp.int32, (_QW, _KW), 1) // 64 - 1
        valid = (kc == qc) | (kc == qc - 1)
        dots = jnp.where(tq < tk, -1e9, dots)
        dots = jnp.where(tq == tk, -1e5, dots)
        dots = jnp.where(valid, dots, -1e30)
        m = jnp.max(dots, axis=1, keepdims=True)
        ex = jnp.exp(dots - m)
        se = jnp.sum(ex, axis=1, keepdims=True)
        lse = m + jnp.log(se)
        o = lax.dot_general(ex / se, vv, (((1,), (0,)), ((), ())),
                            preferred_element_type=jnp.float32)
        att_ref[0, pl.ds(b0, _QW), :] = jnp.concatenate(
            [o, jnp.broadcast_to(lse, (_QW, DH))], axis=1)
        return 0
    lax.fori_loop(0, NCH // _CG, body, 0)


def _attn(sorted_rows):
    return pl.pallas_call(
        _attn_body,
        grid=(H,),
        in_specs=[
            pl.BlockSpec((1, L, 256), lambda i: (i, 0, 0)),
        ],
        out_specs=pl.BlockSpec((1, L, 128), lambda i: (i, 0, 0)),
        out_shape=jax.ShapeDtypeStruct((H, L, 128), jnp.float32),
    )(sorted_rows)


def _scatter_body(att_hbm, stick_hbm, und_hbm, sk, iv, rows, sem):
    wid = lax.axis_index("s") * 2 + lax.axis_index("c")
    _JL = L // 2

    @pl.when(wid < _NJOB)
    def _():
        h = wid // 2
        base = h * L + (wid % 2) * _JL

        def chunk(cix, _):
            off = base + cix * _GC
            pltpu.sync_copy(stick_hbm.at[pl.ds(off, _GC)], sk)

            def body16(i, _):
                s = sk[pl.ds(i * 16, 16)]
                iv[pl.ds(i * 16, 16)] = s + h * L
                return 0
            lax.fori_loop(0, _GC // 16, body16, 0)
            pltpu.sync_copy(att_hbm.at[pl.ds(off, _GC)], rows)
            pltpu.async_copy(rows, und_hbm.at[iv], sem).wait()
            return 0
        lax.fori_loop(0, _JL // _GC, chunk, 0)


def _scatter_sc(att_flat, stick_flat):
    if "scatter" not in _sc_cache:
        mesh = plsc.VectorSubcoreMesh(core_axis_name="c", subcore_axis_name="s")
        _sc_cache["scatter"] = pl.kernel(
            _scatter_body,
            out_type=[jax.ShapeDtypeStruct((H * L, 128), jnp.float32)],
            mesh=mesh,
            compiler_params=_SC_PARAMS,
            scratch_types=[
                pltpu.VMEM((_GC,), jnp.int32),
                pltpu.VMEM((_GC,), jnp.int32),
                pltpu.VMEM((_GC, 128), jnp.float32),
                pltpu.SemaphoreType.DMA,
            ],
        )
    return _sc_cache["scatter"](att_flat, stick_flat)


def _layer_norm_rows(x, g, b):
    mu = jnp.mean(x, axis=-1, keepdims=True)
    xc = x - mu
    var = jnp.mean(xc * xc, axis=-1, keepdims=True)
    return xc / jnp.sqrt(var + 1e-6) * g + b


def _ffn_body(und_ref, x_ref, wo_ref, w1_ref, b1_ref, w2_ref, b2_ref,
              g1_ref, be1_ref, g2_ref, be2_ref, out_ref):
    pieces = []
    for h in range(H):
        u0 = und_ref[h, 0, :, :]
        u1 = und_ref[h, 1, :, :]
        o0 = u0[:, 0:DH]
        o1 = u1[:, 0:DH]
        l0 = u0[:, DH:128]
        l1 = u1[:, DH:128]
        m = jnp.maximum(l0, l1)
        e0 = jnp.exp(l0 - m)
        e1 = jnp.exp(l1 - m)
        pieces.append((e0 * o0 + e1 * o1) / (e0 + e1))
    attn = jnp.concatenate(pieces, axis=1)
    xb = x_ref[...]
    ao = jnp.dot(attn, wo_ref[...], preferred_element_type=jnp.float32) + xb
    h1 = _layer_norm_rows(ao, g1_ref[...], be1_ref[...])
    f = jnp.maximum(
        jnp.dot(h1, w1_ref[...], preferred_element_type=jnp.float32)
        + b1_ref[...], 0.0)
    f2 = jnp.dot(f, w2_ref[...], preferred_element_type=jnp.float32) \
        + b2_ref[...]
    out_ref[...] = _layer_norm_rows(f2 + h1, g2_ref[...], be2_ref[...])


def _ffn(und, x, wo, w1, b1, w2, b2, g1, be1, g2, be2):
    return pl.pallas_call(
        _ffn_body,
        grid=(T // BT,),
        in_specs=[
            pl.BlockSpec((H, 2, BT, 128), lambda i: (0, 0, i, 0)),
            pl.BlockSpec((BT, D), lambda i: (i, 0)),
            pl.BlockSpec((D, D), lambda i: (0, 0)),
            pl.BlockSpec((D, 3072), lambda i: (0, 0)),
            pl.BlockSpec((1, 3072), lambda i: (0, 0)),
            pl.BlockSpec((3072, D), lambda i: (0, 0)),
            pl.BlockSpec((1, D), lambda i: (0, 0)),
            pl.BlockSpec((1, D), lambda i: (0, 0)),
            pl.BlockSpec((1, D), lambda i: (0, 0)),
            pl.BlockSpec((1, D), lambda i: (0, 0)),
            pl.BlockSpec((1, D), lambda i: (0, 0)),
        ],
        out_specs=pl.BlockSpec((BT, D), lambda i: (i, 0)),
        out_shape=jax.ShapeDtypeStruct((T, D), jnp.float32),
    )(und, x, wo, w1, b1, w2, b2, g1, be1, g2, be2)


def kernel(enc_input, Wqk, Wv, Wo, rotations, ln1_g, ln1_b, W1, b1, W2, b2,
           ln2_g, ln2_b):
    x = enc_input.reshape(T, D)
    rotc = jnp.concatenate(
        [rotations.transpose(1, 0, 2), -rotations.transpose(1, 0, 2)],
        axis=-1)

    qkv, bkt = _proj(x, Wqk, Wv, rotc)
    (stick,) = _sort_sc(bkt.reshape(H, L))
    qkv_flat = qkv.reshape(H * T, 256)
    stick_flat = stick.reshape(H * L)
    (sorted_flat,) = _gather_sc(qkv_flat, stick_flat)
    att = _attn(sorted_flat.reshape(H, L, 256))
    (und_flat,) = _scatter_sc(att.reshape(H * L, 128), stick_flat)
    und = und_flat.reshape(H, 2, T, 128)
    out = _ffn(und, x, Wo, W1, b1.reshape(1, 3072), W2, b2.reshape(1, D),
               ln1_g.reshape(1, D), ln1_b.reshape(1, D),
               ln2_g.reshape(1, D), ln2_b.reshape(1, D))
    return out.reshape(1, T, D)


# speedup vs baseline: 1.1449x; 1.1331x over previous
import functools

import jax
import jax.numpy as jnp
from jax import lax
from jax.experimental import pallas as pl
from jax.experimental.pallas import tpu as pltpu
from jax.experimental.pallas import tpu_sc as plsc

D = 768
H = 12
DH = 64
T = 8192
L = 2 * T
NCH = L // 64
NBKT = T // 64
NBIN = 2 * NBKT
SEG = L // 16
BT = 512

_SC_PARAMS = pltpu.CompilerParams(needs_layout_passes=False)
_sc_cache = {}


def _proj_body(x_ref, wqk_ref, wv_ref, rotc_ref, qkv_ref, bkt_ref):
    xb = x_ref[...]
    qk = jnp.dot(xb, wqk_ref[...], preferred_element_type=jnp.float32)
    v = jnp.dot(xb, wv_ref[...], preferred_element_type=jnp.float32)
    i = pl.program_id(0)
    tcol = (i * BT
            + lax.broadcasted_iota(jnp.int32, (BT, DH), 0)).astype(jnp.float32)
    zcol = jnp.zeros((BT, DH), jnp.float32)
    cols = []
    for h in range(H):
        qkh = qk[:, h * DH:(h + 1) * DH]
        qkv_ref[h, :, :] = jnp.concatenate(
            [qkh, v[:, h * DH:(h + 1) * DH], tcol, zcol], axis=1)
        for r in range(2):
            rot = jnp.dot(qkh, rotc_ref[r], preferred_element_type=jnp.float32)
            mx = jnp.max(rot, axis=1, keepdims=True)
            iota = lax.broadcasted_iota(
                jnp.int32, rot.shape, 1).astype(jnp.float32)
            cand = jnp.where(rot >= mx, iota, float(NBKT))
            cols.append(jnp.min(cand, axis=1, keepdims=True))
    bmat = jnp.transpose(jnp.concatenate(cols, axis=1))
    roff = lax.broadcasted_iota(jnp.int32, (2 * H, BT), 0) % 2
    bkt_ref[...] = bmat.astype(jnp.int32) + roff * NBKT


def _proj(x, wqk, wv, rotc):
    return pl.pallas_call(
        _proj_body,
        grid=(T // BT,),
        in_specs=[
            pl.BlockSpec((BT, D), lambda i: (i, 0)),
            pl.BlockSpec((D, D), lambda i: (0, 0)),
            pl.BlockSpec((D, D), lambda i: (0, 0)),
            pl.BlockSpec((2, DH, NBKT), lambda i: (0, 0, 0)),
        ],
        out_specs=[
            pl.BlockSpec((H, BT, 256), lambda i: (0, i, 0)),
            pl.BlockSpec((2 * H, BT), lambda i: (0, i)),
        ],
        out_shape=[
            jax.ShapeDtypeStruct((H, T, 256), jnp.float32),
            jax.ShapeDtypeStruct((2 * H, T), jnp.int32),
        ],
    )(x, wqk, wv, rotc)


def _sort_body(bkt_hbm, stick_hbm, bv, sv, cv, t16):
    wid = lax.axis_index("s") * 2 + lax.axis_index("c")
    lane = lax.iota(jnp.int32, 16)

    @pl.when(wid < H)
    def _():
        pltpu.sync_copy(bkt_hbm.at[wid], bv)

        def zero(i, _):
            cv[pl.ds(i * 16, 16)] = jnp.zeros((16,), jnp.int32)
            return 0
        lax.fori_loop(0, NBIN, zero, 0)

        def hist(i, _):
            b = plsc.load_gather(bv, [lane * SEG + i])
            plsc.addupdate_scatter(cv, [lane * NBIN + b],
                                   jnp.ones((16,), jnp.int32))
            return 0
        lax.fori_loop(0, SEG, hist, 0)

        def prefix(vb, carry):
            c = plsc.load_gather(cv, [lane * NBIN + vb])
            s = c
            for sh in (1, 2, 4, 8):
                t16[...] = s
                g = plsc.load_gather(t16, [jnp.maximum(lane - sh, 0)])
                s = s + jnp.where(lane >= sh, g, jnp.zeros((16,), jnp.int32))
            t16[...] = s
            tot = plsc.load_gather(t16, [jnp.full((16,), 15, jnp.int32)])
            plsc.store_scatter(cv, [lane * NBIN + vb], s - c + carry)
            return carry + tot
        lax.fori_loop(0, NBIN, prefix, jnp.zeros((16,), jnp.int32))

        def scat(i, _):
            idx_el = lane * SEG + i
            b = plsc.load_gather(bv, [idx_el])
            cur = plsc.load_gather(cv, [lane * NBIN + b])
            plsc.store_scatter(sv, [cur], idx_el)
            plsc.store_scatter(cv, [lane * NBIN + b], cur + 1)
            return 0
        lax.fori_loop(0, SEG, scat, 0)

        pltpu.sync_copy(sv, stick_hbm.at[wid])


def _sort_sc(bkt):
    if "sort" not in _sc_cache:
        mesh = plsc.VectorSubcoreMesh(core_axis_name="c", subcore_axis_name="s")
        _sc_cache["sort"] = pl.kernel(
            _sort_body,
            out_type=[jax.ShapeDtypeStruct((H, L), jnp.int32)],
            mesh=mesh,
            compiler_params=_SC_PARAMS,
            scratch_types=[
                pltpu.VMEM((L,), jnp.int32),
                pltpu.VMEM((L,), jnp.int32),
                pltpu.VMEM((16 * NBIN,), jnp.int32),
                pltpu.VMEM((16,), jnp.int32),
            ],
        )
    return _sc_cache["sort"](bkt)


_GC = 256
_NJOB = 2 * H


def _gather_body(qkv_hbm, stick_hbm, sorted_hbm, sk, iv, rows, sem):
    wid = lax.axis_index("s") * 2 + lax.axis_index("c")
    _JL = L // 2

    @pl.when(wid < _NJOB)
    def _():
        h = wid // 2
        base = h * L + (wid % 2) * _JL

        def chunk(cix, _):
            off = base + cix * _GC
            pltpu.sync_copy(stick_hbm.at[pl.ds(off, _GC)], sk)

            def body16(i, _):
                s = sk[pl.ds(i * 16, 16)]
                iv[pl.ds(i * 16, 16)] = (s & (T - 1)) + h * T
                return 0
            lax.fori_loop(0, _GC // 16, body16, 0)
            pltpu.async_copy(qkv_hbm.at[iv], rows, sem).wait()
            pltpu.sync_copy(rows, sorted_hbm.at[pl.ds(off, _GC)])
            return 0
        lax.fori_loop(0, _JL // _GC, chunk, 0)


def _gather_sc(qkv_flat, stick_flat):
    if "gather" not in _sc_cache:
        mesh = plsc.VectorSubcoreMesh(core_axis_name="c", subcore_axis_name="s")
        _sc_cache["gather"] = pl.kernel(
            _gather_body,
            out_type=[
                jax.ShapeDtypeStruct((H * L, 256), jnp.float32),
            ],
            mesh=mesh,
            compiler_params=_SC_PARAMS,
            scratch_types=[
                pltpu.VMEM((_GC,), jnp.int32),
                pltpu.VMEM((_GC,), jnp.int32),
                pltpu.VMEM((_GC, 256), jnp.float32),
                pltpu.SemaphoreType.DMA,
            ],
        )
    return _sc_cache["gather"](qkv_flat, stick_flat)


_CG = 4
_QW = _CG * 64
_KW = _QW + 64


def _attn_body(sorted_ref, att_ref):
    qc0 = lax.broadcasted_iota(jnp.int32, (_QW, _KW), 0) // 64
    kc0 = lax.broadcasted_iota(jnp.int32, (_QW, _KW), 1) // 64 - 1
    invalid0 = jnp.logical_not((kc0 == qc0) | (kc0 == qc0 - 1))

    def group(g):
        b0 = g * _QW
        p0 = (b0 + L - 64) % L
        cur = sorted_ref[0, pl.ds(b0, _QW), :]
        prv = sorted_ref[0, pl.ds(p0, 64), :]
        win = jnp.concatenate([prv, cur], axis=0)
        q = cur[:, 0:DH]
        kk = win[:, 0:DH]
        vv = win[:, DH:128]
        n2 = jnp.sum(kk * kk, axis=1, keepdims=True)
        nk = kk * ((float(DH) ** -0.5) / (jnp.sqrt(n2) + 1e-9))
        dots = lax.dot_general(q, nk, (((1,), (1,)), ((), ())),
                               preferred_element_type=jnp.float32)
        tq64 = cur[:, 128:192]
        tq = jnp.concatenate([tq64] * (_KW // DH), axis=1)
        tk = jnp.concatenate(
            [jnp.transpose(win[:, 128:192])] * (_QW // DH), axis=0)
        dots = jnp.where(tq < tk, -1e9, dots)
        dots = jnp.where(tq == tk, -1e5, dots)
        dots = jnp.where(invalid0, -1e30, dots)
        m = jnp.max(dots, axis=1, keepdims=True)
        ex = jnp.exp(dots - m)
        se = jnp.sum(ex, axis=1, keepdims=True)
        probs = ex / se
        lse = m + jnp.log(se)
        o = lax.dot_general(probs, vv, (((1,), (0,)), ((), ())),
                            preferred_element_type=jnp.float32)
        att_ref[0, pl.ds(b0, _QW), :] = jnp.concatenate(
            [o, jnp.zeros((_QW, DH), jnp.float32) + lse], axis=1)

    def body(g2, _):
        group(g2 * 2)
        group(g2 * 2 + 1)
        return 0
    lax.fori_loop(0, NCH // _CG // 2, body, 0)


def _attn(sorted_rows):
    return pl.pallas_call(
        _attn_body,
        grid=(H,),
        in_specs=[
            pl.BlockSpec((1, L, 256), lambda i: (i, 0, 0)),
        ],
        out_specs=pl.BlockSpec((1, L, 128), lambda i: (i, 0, 0)),
        out_shape=jax.ShapeDtypeStruct((H, L, 128), jnp.float32),
    )(sorted_rows)


def _scatter_body(att_hbm, stick_hbm, und_hbm, sk, iv, rows, sem):
    wid = lax.axis_index("s") * 2 + lax.axis_index("c")
    _JL = L // 2

    @pl.when(wid < _NJOB)
    def _():
        h = wid // 2
        base = h * L + (wid % 2) * _JL

        def chunk(cix, _):
            off = base + cix * _GC
            pltpu.sync_copy(stick_hbm.at[pl.ds(off, _GC)], sk)

            def body16(i, _):
                s = sk[pl.ds(i * 16, 16)]
                iv[pl.ds(i * 16, 16)] = s + h * L
                return 0
            lax.fori_loop(0, _GC // 16, body16, 0)
            pltpu.sync_copy(att_hbm.at[pl.ds(off, _GC)], rows)
            pltpu.async_copy(rows, und_hbm.at[iv], sem).wait()
            return 0
        lax.fori_loop(0, _JL // _GC, chunk, 0)


def _scatter_sc(att_flat, stick_flat):
    if "scatter" not in _sc_cache:
        mesh = plsc.VectorSubcoreMesh(core_axis_name="c", subcore_axis_name="s")
        _sc_cache["scatter"] = pl.kernel(
            _scatter_body,
            out_type=[jax.ShapeDtypeStruct((H * L, 128), jnp.float32)],
            mesh=mesh,
            compiler_params=_SC_PARAMS,
            scratch_types=[
                pltpu.VMEM((_GC,), jnp.int32),
                pltpu.VMEM((_GC,), jnp.int32),
                pltpu.VMEM((_GC, 128), jnp.float32),
                pltpu.SemaphoreType.DMA,
            ],
        )
    return _sc_cache["scatter"](att_flat, stick_flat)


def _layer_norm_rows(x, g, b):
    mu = jnp.mean(x, axis=-1, keepdims=True)
    xc = x - mu
    var = jnp.mean(xc * xc, axis=-1, keepdims=True)
    return xc / jnp.sqrt(var + 1e-6) * g + b


def _ffn_body(und_ref, x_ref, wo_ref, w1_ref, b1_ref, w2_ref, b2_ref,
              g1_ref, be1_ref, g2_ref, be2_ref, out_ref):
    pieces = []
    for h in range(H):
        u0 = und_ref[h, 0, :, :]
        u1 = und_ref[h, 1, :, :]
        o0 = u0[:, 0:DH]
        o1 = u1[:, 0:DH]
        l0 = u0[:, DH:128]
        l1 = u1[:, DH:128]
        m = jnp.maximum(l0, l1)
        e0 = jnp.exp(l0 - m)
        e1 = jnp.exp(l1 - m)
        pieces.append((e0 * o0 + e1 * o1) / (e0 + e1))
    attn = jnp.concatenate(pieces, axis=1)
    xb = x_ref[...]
    ao = jnp.dot(attn, wo_ref[...], preferred_element_type=jnp.float32) + xb
    h1 = _layer_norm_rows(ao, g1_ref[...], be1_ref[...])
    f = jnp.maximum(
        jnp.dot(h1, w1_ref[...], preferred_element_type=jnp.float32)
        + b1_ref[...], 0.0)
    f2 = jnp.dot(f, w2_ref[...], preferred_element_type=jnp.float32) \
        + b2_ref[...]
    out_ref[...] = _layer_norm_rows(f2 + h1, g2_ref[...], be2_ref[...])


def _ffn(und, x, wo, w1, b1, w2, b2, g1, be1, g2, be2):
    return pl.pallas_call(
        _ffn_body,
        grid=(T // BT,),
        in_specs=[
            pl.BlockSpec((H, 2, BT, 128), lambda i: (0, 0, i, 0)),
            pl.BlockSpec((BT, D), lambda i: (i, 0)),
            pl.BlockSpec((D, D), lambda i: (0, 0)),
            pl.BlockSpec((D, 3072), lambda i: (0, 0)),
            pl.BlockSpec((1, 3072), lambda i: (0, 0)),
            pl.BlockSpec((3072, D), lambda i: (0, 0)),
            pl.BlockSpec((1, D), lambda i: (0, 0)),
            pl.BlockSpec((1, D), lambda i: (0, 0)),
            pl.BlockSpec((1, D), lambda i: (0, 0)),
            pl.BlockSpec((1, D), lambda i: (0, 0)),
            pl.BlockSpec((1, D), lambda i: (0, 0)),
        ],
        out_specs=pl.BlockSpec((BT, D), lambda i: (i, 0)),
        out_shape=jax.ShapeDtypeStruct((T, D), jnp.float32),
    )(und, x, wo, w1, b1, w2, b2, g1, be1, g2, be2)


def kernel(enc_input, Wqk, Wv, Wo, rotations, ln1_g, ln1_b, W1, b1, W2, b2,
           ln2_g, ln2_b):
    x = enc_input.reshape(T, D)
    rotc = jnp.concatenate(
        [rotations.transpose(1, 0, 2), -rotations.transpose(1, 0, 2)],
        axis=-1)

    qkv, bkt = _proj(x, Wqk, Wv, rotc)
    (stick,) = _sort_sc(bkt.reshape(H, L))
    qkv_flat = qkv.reshape(H * T, 256)
    stick_flat = stick.reshape(H * L)
    (sorted_flat,) = _gather_sc(qkv_flat, stick_flat)
    att = _attn(sorted_flat.reshape(H, L, 256))
    (und_flat,) = _scatter_sc(att.reshape(H * L, 128), stick_flat)
    und = und_flat.reshape(H, 2, T, 128)
    out = _ffn(und, x, Wo, W1, b1.reshape(1, 3072), W2, b2.reshape(1, D),
               ln1_g.reshape(1, D), ln1_b.reshape(1, D),
               ln2_g.reshape(1, D), ln2_b.reshape(1, D))
    return out.reshape(1, T, D)
